# baseline (speedup 1.0000x reference)
import functools

import jax
import jax.numpy as jnp
from jax import lax
from jax.experimental import pallas as pl
from jax.experimental.pallas import tpu as pltpu
from jax.experimental.pallas import tpu_sc as plsc


def _make_gather_kernel(B, D, n_cores, n_subcores):
    nw = n_cores * n_subcores
    b_per_w = B // nw
    mesh = plsc.VectorSubcoreMesh(core_axis_name="c", subcore_axis_name="s")

    @functools.partial(
        pl.kernel,
        mesh=mesh,
        compiler_params=pltpu.CompilerParams(use_tc_tiling_on_sc=False),
        out_type=(
            jax.ShapeDtypeStruct((B, D), jnp.float32),
            jax.ShapeDtypeStruct((B, D), jnp.float32),
        ),
        scratch_types=[
            pltpu.VMEM((b_per_w,), jnp.int32),
            pltpu.VMEM((b_per_w, D), jnp.float32),
            pltpu.VMEM((b_per_w, D), jnp.float32),
            pltpu.SemaphoreType.DMA,
            pltpu.SemaphoreType.DMA,
        ],
    )
    def gather_kernel(idx_hbm, mu_hbm, ls_hbm, mu_out, ls_out,
                      idx_v, mu_v, ls_v, sem_mu, sem_ls):
        wid = lax.axis_index("s") * n_cores + lax.axis_index("c")
        base = wid * b_per_w
        pltpu.sync_copy(idx_hbm.at[pl.ds(base, b_per_w)], idx_v)
        cp_mu = pltpu.async_copy(mu_hbm.at[idx_v], mu_v, sem_mu)
        cp_ls = pltpu.async_copy(ls_hbm.at[idx_v], ls_v, sem_ls)
        cp_mu.wait()
        pltpu.sync_copy(mu_v, mu_out.at[pl.ds(base, b_per_w)])
        cp_ls.wait()
        pltpu.sync_copy(ls_v, ls_out.at[pl.ds(base, b_per_w)])

    return gather_kernel


def kernel(indices, mu, log_sigma):
    B = indices.shape[0]
    D = mu.shape[1]
    info = plsc.get_sparse_core_info()
    gather = _make_gather_kernel(B, D, info.num_cores, info.num_subcores)
    return gather(indices.astype(jnp.int32), mu, log_sigma)


# speedup vs baseline: 1.1772x; 1.1772x over previous
import functools

import jax
import jax.numpy as jnp
from jax import lax
from jax.experimental import pallas as pl
from jax.experimental.pallas import tpu as pltpu
from jax.experimental.pallas import tpu_sc as plsc

_SUB = 8


def _make_gather_kernel(B, D, n_cores, n_subcores):
    nw = n_cores * n_subcores
    b_per_w = B // nw

    mesh = plsc.VectorSubcoreMesh(core_axis_name="c", subcore_axis_name="s")

    @functools.partial(
        pl.kernel,
        mesh=mesh,
        out_type=(
            jax.ShapeDtypeStruct((B // _SUB, _SUB, D), jnp.float32),
            jax.ShapeDtypeStruct((B // _SUB, _SUB, D), jnp.float32),
        ),
        scratch_types=[
            pltpu.VMEM((b_per_w,), jnp.int32),
            pltpu.SemaphoreType.DMA,
        ],
    )
    def gather_kernel(idx_hbm, mu_hbm, ls_hbm, mu_out, ls_out,
                      idx_s, sem):
        wid = lax.axis_index("s") * n_cores + lax.axis_index("c")
        base = pl.multiple_of(wid * b_per_w, b_per_w)
        pltpu.sync_copy(idx_hbm.at[pl.ds(base, b_per_w)], idx_s)
        qbase = wid * (b_per_w // _SUB)
        grp = 16
        n_grp = b_per_w // grp

        def row_slices(g, j):
            v = idx_s[pl.ds(g * grp, grp)]
            tv = lax.div(v, _SUB)
            sv = lax.rem(v, _SUB)
            t = tv[j]
            s = sv[j]
            q = qbase + g * (grp // _SUB) + j // _SUB
            m = j % _SUB
            return t, s, q, m

        def issue(g, carry):
            for j in range(grp):
                t, s, q, m = row_slices(g, j)
                pltpu.async_copy(mu_hbm.at[t, s], mu_out.at[q, m], sem)
                pltpu.async_copy(ls_hbm.at[t, s], ls_out.at[q, m], sem)
            return carry

        def drain(g, carry):
            for j in range(grp):
                t, s, q, m = row_slices(g, j)
                pltpu.make_async_copy(
                    mu_hbm.at[t, s], mu_out.at[q, m], sem).wait()
                pltpu.make_async_copy(
                    ls_hbm.at[t, s], ls_out.at[q, m], sem).wait()
            return carry

        lax.fori_loop(0, n_grp, issue, 0)
        lax.fori_loop(0, n_grp, drain, 0)

    return gather_kernel


def kernel(indices, mu, log_sigma):
    B = indices.shape[0]
    N, D = mu.shape
    info = plsc.get_sparse_core_info()
    gather = _make_gather_kernel(B, D, info.num_cores, info.num_subcores)
    mu3 = mu.reshape(N // _SUB, _SUB, D)
    ls3 = log_sigma.reshape(N // _SUB, _SUB, D)
    mu_out, ls_out = gather(indices.astype(jnp.int32), mu3, ls3)
    return (mu_out.reshape(B, D), ls_out.reshape(B, D))
